# baseline (speedup 1.0000x reference)
import functools

import jax
import jax.numpy as jnp
from jax import lax
from jax.experimental import pallas as pl
from jax.experimental.pallas import tpu as pltpu
from jax.experimental.pallas import tpu_sc as plsc

N_NODE = 10000
E_TOT = 320000
C = 128

_NB = 1000
_NW = 32

_PHASES = 2
_EH = E_TOT // _PHASES

_sc_mesh = functools.partial(
    plsc.VectorSubcoreMesh, core_axis_name="c", subcore_axis_name="s")


def _silu(x):
    return x * jax.nn.sigmoid(x)


def _wid():
    return lax.axis_index("s") * 2 + lax.axis_index("c")


def _make_gather(e_tot, ch):
    epw = e_tot // _NW
    nch = epw // ch
    assert epw * _NW == e_tot and nch * ch == epw and ch % 8 == 0

    def body(a_hbm, b_hbm, es_hbm, et_hbm, a_out, b_out,
             bufs0, bufs1, gsem0, gsem1, osem0, osem1):
        tbase = _wid() * epw

        def fire_g(base, bufs, gsem):
            idx_s, idx_t, ab, bb = bufs
            pltpu.sync_copy(es_hbm.at[pl.ds(base, ch)], idx_s)
            pltpu.sync_copy(et_hbm.at[pl.ds(base, ch)], idx_t)
            pltpu.async_copy(a_hbm.at[idx_s], ab, gsem)
            pltpu.async_copy(b_hbm.at[idx_t], bb, gsem)

        def wait_g(bufs, sem):
            _, _, ab, bb = bufs
            pltpu.make_async_copy(a_hbm.at[pl.ds(0, ch)], ab, sem).wait()
            pltpu.make_async_copy(b_hbm.at[pl.ds(0, ch)], bb, sem).wait()

        def fire_o(base, bufs, sem):
            _, _, ab, bb = bufs
            pltpu.async_copy(ab, a_out.at[pl.ds(base, ch)], sem)
            pltpu.async_copy(bb, b_out.at[pl.ds(base, ch)], sem)

        def wait_o(bufs, sem):
            _, _, ab, bb = bufs
            pltpu.make_async_copy(ab, a_out.at[pl.ds(0, ch)], sem).wait()
            pltpu.make_async_copy(bb, b_out.at[pl.ds(0, ch)], sem).wait()

        fire_g(tbase, bufs0, gsem0)

        def loop(j, carry):
            base0 = tbase + (2 * j) * ch
            base1 = base0 + ch
            base2 = base0 + 2 * ch

            @pl.when(j > 0)
            def _():
                wait_o(bufs1, osem1)

            @pl.when(2 * j + 1 < nch)
            def _():
                fire_g(base1, bufs1, gsem1)

            wait_g(bufs0, gsem0)
            fire_o(base0, bufs0, osem0)

            @pl.when(2 * j + 2 < nch)
            def _():
                wait_o(bufs0, osem0)
                fire_g(base2, bufs0, gsem0)

            @pl.when(2 * j + 1 < nch)
            def _():
                wait_g(bufs1, gsem1)
                fire_o(base1, bufs1, osem1)

            return carry

        lax.fori_loop(0, (nch + 1) // 2, loop, 0)
        wait_o(bufs0, osem0)
        if nch % 2 == 0:
            wait_o(bufs1, osem1)

    bufset = lambda: [
        pltpu.VMEM((ch,), jnp.int32),
        pltpu.VMEM((ch,), jnp.int32),
        pltpu.VMEM((ch, 2 * C), jnp.float32),
        pltpu.VMEM((ch, 2 * C), jnp.float32),
    ]
    return pl.kernel(
        body, mesh=_sc_mesh(),
        out_type=[
            jax.ShapeDtypeStruct((e_tot, 2 * C), jnp.float32),
            jax.ShapeDtypeStruct((e_tot, 2 * C), jnp.float32),
        ],
        scratch_types=[
            bufset(), bufset(),
            pltpu.SemaphoreType.DMA, pltpu.SemaphoreType.DMA,
            pltpu.SemaphoreType.DMA, pltpu.SemaphoreType.DMA,
        ],
    )


def _make_geo(e_tot):
    epw = e_tot // _NW
    assert epw * _NW == e_tot and epw % 8 == 0
    ngrp = (epw + 15) // 16

    def body(psx_h, psy_h, psz_h, ptx_h, pty_h, ptz_h, es_hbm, et_hbm,
             sq_out, ptabs, idx_s, idx_t, sqb):
        tbase = _wid() * epw

        psx_t, psy_t, psz_t, ptx_t, pty_t, ptz_t = ptabs
        pltpu.sync_copy(psx_h, psx_t)
        pltpu.sync_copy(psy_h, psy_t)
        pltpu.sync_copy(psz_h, psz_t)
        pltpu.sync_copy(ptx_h, ptx_t)
        pltpu.sync_copy(pty_h, pty_t)
        pltpu.sync_copy(ptz_h, ptz_t)

        pltpu.sync_copy(es_hbm.at[pl.ds(tbase, epw)], idx_s)
        pltpu.sync_copy(et_hbm.at[pl.ds(tbase, epw)], idx_t)

        def grp(g, carry):
            sl = pl.ds(jnp.minimum(g * 16, epw - 16), 16)
            i_s = idx_s[sl]
            i_t = idx_t[sl]
            dx = (plsc.load_gather(ptx_t, [i_t])
                  - plsc.load_gather(psx_t, [i_s]))
            dy = (plsc.load_gather(pty_t, [i_t])
                  - plsc.load_gather(psy_t, [i_s]))
            dz = (plsc.load_gather(ptz_t, [i_t])
                  - plsc.load_gather(psz_t, [i_s]))
            sqb[sl] = dx * dx + dy * dy + dz * dz
            return carry

        lax.fori_loop(0, ngrp, grp, 0)
        pltpu.sync_copy(sqb, sq_out.at[pl.ds(tbase, epw)])

    return pl.kernel(
        body, mesh=_sc_mesh(),
        compiler_params=pltpu.CompilerParams(needs_layout_passes=False),
        out_type=[jax.ShapeDtypeStruct((e_tot,), jnp.float32)],
        scratch_types=[
            [pltpu.VMEM((N_NODE,), jnp.float32) for _ in range(6)],
            pltpu.VMEM((epw,), jnp.int32),
            pltpu.VMEM((epw,), jnp.int32),
            pltpu.VMEM((epw,), jnp.float32),
        ],
    )


def _make_scatter(e_tot, ch):
    epw = e_tot // _NW
    nch = epw // ch
    assert nch * ch == epw and ch % 8 == 0

    def body(msg_hbm, et_hbm, init_hbm, agg_out,
             bufs0, bufs1, agg_acc, lsem0, lsem1):
        c = lax.axis_index("c")
        s = lax.axis_index("s")
        tbase = (s * 2 + c) * epw

        @pl.when(s == 0)
        def _init():
            pltpu.sync_copy(init_hbm.at[c], agg_acc)

        def fire_l(base, bufs, sem):
            idx_t, mb = bufs
            pltpu.sync_copy(et_hbm.at[pl.ds(base, ch)], idx_t)
            pltpu.async_copy(msg_hbm.at[pl.ds(base, ch)], mb, sem)

        def wait_l(bufs, sem):
            pltpu.make_async_copy(msg_hbm.at[pl.ds(0, ch)], bufs[1],
                                  sem).wait()

        def scat(bufs):
            idx_t, mb = bufs
            pltpu.sync_copy(mb, agg_acc.at[idx_t], add=True)

        plsc.subcore_barrier()
        fire_l(tbase, bufs0, lsem0)

        def loop(j, carry):
            base1 = tbase + (2 * j + 1) * ch
            base2 = tbase + (2 * j + 2) * ch

            @pl.when(2 * j + 1 < nch)
            def _():
                fire_l(base1, bufs1, lsem1)

            wait_l(bufs0, lsem0)
            scat(bufs0)

            @pl.when(2 * j + 2 < nch)
            def _():
                fire_l(base2, bufs0, lsem0)

            @pl.when(2 * j + 1 < nch)
            def _():
                wait_l(bufs1, lsem1)
                scat(bufs1)

            return carry

        lax.fori_loop(0, (nch + 1) // 2, loop, 0)
        plsc.subcore_barrier()

        rows = N_NODE // 16
        rbase = s * rows
        pltpu.sync_copy(agg_acc.at[pl.ds(rbase, rows)],
                        agg_out.at[c].at[pl.ds(rbase, rows)])

    bufset = lambda: [
        pltpu.VMEM((ch,), jnp.int32),
        pltpu.VMEM((ch, C), jnp.float32),
    ]
    return pl.kernel(
        body, mesh=_sc_mesh(),
        compiler_params=pltpu.CompilerParams(use_tc_tiling_on_sc=False),
        out_type=[jax.ShapeDtypeStruct((2, N_NODE, C), jnp.float32)],
        scratch_types=[
            bufset(), bufset(),
            pltpu.VMEM_SHARED((N_NODE, C), jnp.float32),
            pltpu.SemaphoreType.DMA, pltpu.SemaphoreType.DMA,
        ],
    )


def _make_vel(e_tot, ch):
    epw = e_tot // _NW
    nch = epw // ch
    assert nch * ch == epw and ch % 8 == 0

    def body(w_hbm, psx_h, psy_h, psz_h, ptx_h, pty_h, ptz_h,
             es_hbm, et_hbm, ivx_h, ivy_h, ivz_h,
             vx_out, vy_out, vz_out,
             bufs0, bufs1, ptabs, accs, lsem0, lsem1):
        c = lax.axis_index("c")
        s = lax.axis_index("s")
        tbase = (s * 2 + c) * epw
        vx_acc, vy_acc, vz_acc = accs

        psx_t, psy_t, psz_t, ptx_t, pty_t, ptz_t = ptabs
        pltpu.sync_copy(psx_h, psx_t)
        pltpu.sync_copy(psy_h, psy_t)
        pltpu.sync_copy(psz_h, psz_t)
        pltpu.sync_copy(ptx_h, ptx_t)
        pltpu.sync_copy(pty_h, pty_t)
        pltpu.sync_copy(ptz_h, ptz_t)

        @pl.when(s == 0)
        def _init():
            pltpu.sync_copy(ivx_h.at[c], vx_acc)
            pltpu.sync_copy(ivy_h.at[c], vy_acc)
            pltpu.sync_copy(ivz_h.at[c], vz_acc)

        def fire_l(base, bufs, sem):
            idx_s, idx_t, wb, xb, yb, zb = bufs
            pltpu.sync_copy(es_hbm.at[pl.ds(base, ch)], idx_s)
            pltpu.sync_copy(et_hbm.at[pl.ds(base, ch)], idx_t)
            pltpu.async_copy(w_hbm.at[pl.ds(base, ch)], wb, sem)

        def wait_l(bufs, sem):
            pltpu.make_async_copy(w_hbm.at[pl.ds(0, ch)], bufs[2],
                                  sem).wait()

        def scat(bufs):
            idx_s, idx_t, wb, xb, yb, zb = bufs
            for g in range((ch + 15) // 16):
                sl = pl.ds(min(g * 16, ch - 16), 16)
                i_s = idx_s[sl]
                i_t = idx_t[sl]
                wv = wb[sl]
                xb[sl] = wv * (plsc.load_gather(ptx_t, [i_t])
                               - plsc.load_gather(psx_t, [i_s]))
                yb[sl] = wv * (plsc.load_gather(pty_t, [i_t])
                               - plsc.load_gather(psy_t, [i_s]))
                zb[sl] = wv * (plsc.load_gather(ptz_t, [i_t])
                               - plsc.load_gather(psz_t, [i_s]))
            pltpu.sync_copy(xb, vx_acc.at[idx_t], add=True)
            pltpu.sync_copy(yb, vy_acc.at[idx_t], add=True)
            pltpu.sync_copy(zb, vz_acc.at[idx_t], add=True)

        plsc.subcore_barrier()
        fire_l(tbase, bufs0, lsem0)

        def loop(j, carry):
            base1 = tbase + (2 * j + 1) * ch
            base2 = tbase + (2 * j + 2) * ch

            @pl.when(2 * j + 1 < nch)
            def _():
                fire_l(base1, bufs1, lsem1)

            wait_l(bufs0, lsem0)
            scat(bufs0)

            @pl.when(2 * j + 2 < nch)
            def _():
                fire_l(base2, bufs0, lsem0)

            @pl.when(2 * j + 1 < nch)
            def _():
                wait_l(bufs1, lsem1)
                scat(bufs1)

            return carry

        lax.fori_loop(0, (nch + 1) // 2, loop, 0)
        plsc.subcore_barrier()

        rows = N_NODE // 10
        rbase = s * rows

        @pl.when(s < 10)
        def _dump():
            pltpu.sync_copy(vx_acc.at[pl.ds(rbase, rows)],
                            vx_out.at[c].at[pl.ds(rbase, rows)])
            pltpu.sync_copy(vy_acc.at[pl.ds(rbase, rows)],
                            vy_out.at[c].at[pl.ds(rbase, rows)])
            pltpu.sync_copy(vz_acc.at[pl.ds(rbase, rows)],
                            vz_out.at[c].at[pl.ds(rbase, rows)])

    bufset = lambda: [
        pltpu.VMEM((ch,), jnp.int32),
        pltpu.VMEM((ch,), jnp.int32),
        pltpu.VMEM((ch,), jnp.float32),
        pltpu.VMEM((ch,), jnp.float32),
        pltpu.VMEM((ch,), jnp.float32),
        pltpu.VMEM((ch,), jnp.float32),
    ]
    return pl.kernel(
        body, mesh=_sc_mesh(),
        compiler_params=pltpu.CompilerParams(
            use_tc_tiling_on_sc=False, needs_layout_passes=False),
        out_type=[
            jax.ShapeDtypeStruct((2, N_NODE), jnp.float32),
            jax.ShapeDtypeStruct((2, N_NODE), jnp.float32),
            jax.ShapeDtypeStruct((2, N_NODE), jnp.float32),
        ],
        scratch_types=[
            bufset(), bufset(),
            [pltpu.VMEM((N_NODE,), jnp.float32) for _ in range(6)],
            [pltpu.VMEM_SHARED((N_NODE,), jnp.float32) for _ in range(3)],
            pltpu.SemaphoreType.DMA, pltpu.SemaphoreType.DMA,
        ],
    )


def _pre_body(h_src_ref, h_tgt_ref, t_emb_ref, w1s_ref, w1t_ref, w1e_ref,
              b1_ref, a_ref, b_ref):
    a_ref[...] = jnp.dot(h_src_ref[...], w1s_ref[...],
                         preferred_element_type=jnp.float32)
    b_ref[...] = (jnp.dot(h_tgt_ref[...], w1t_ref[...],
                          preferred_element_type=jnp.float32)
                  + jnp.dot(t_emb_ref[...], w1e_ref[...],
                            preferred_element_type=jnp.float32)
                  + b1_ref[...])


def _node_pre(h_src, h_tgt, t_emb, w1s, w1t, w1e, b1):
    grid = N_NODE // _NB
    return pl.pallas_call(
        _pre_body,
        grid=(grid,),
        in_specs=[
            pl.BlockSpec((_NB, C), lambda i: (i, 0)),
            pl.BlockSpec((_NB, C), lambda i: (i, 0)),
            pl.BlockSpec((_NB, 32), lambda i: (i, 0)),
            pl.BlockSpec((C, 2 * C), lambda i: (0, 0)),
            pl.BlockSpec((C, 2 * C), lambda i: (0, 0)),
            pl.BlockSpec((32, 2 * C), lambda i: (0, 0)),
            pl.BlockSpec((1, 2 * C), lambda i: (0, 0)),
        ],
        out_specs=[
            pl.BlockSpec((_NB, 2 * C), lambda i: (i, 0)),
            pl.BlockSpec((_NB, 2 * C), lambda i: (i, 0)),
        ],
        out_shape=[
            jax.ShapeDtypeStruct((N_NODE, 2 * C), jnp.float32),
            jax.ShapeDtypeStruct((N_NODE, 2 * C), jnp.float32),
        ],
    )(h_src, h_tgt, t_emb, w1s, w1t, w1e, b1)


def _make_edge_mlp(e_tot, eb):
    grid = e_tot // eb
    assert grid * eb == e_tot

    def body(a_ref, b_ref, sq_ref, w1d_ref, w2_ref, b2_ref, wc1_ref,
             bc1_ref, wc2_ref, bc2_ref, msg_ref, w_ref):
        sq2 = sq_ref[...].reshape(1, eb)
        sq_term = lax.dot_general(sq2, w1d_ref[...], (((0,), (0,)), ((), ())),
                                  preferred_element_type=jnp.float32)
        pre = a_ref[...] + b_ref[...] + sq_term
        h1 = _silu(pre)
        msg = _silu(jnp.dot(h1, w2_ref[...],
                            preferred_element_type=jnp.float32) + b2_ref[...])
        t1 = _silu(jnp.dot(msg, wc1_ref[...],
                           preferred_element_type=jnp.float32) + bc1_ref[...])
        w = jnp.tanh(lax.dot_general(wc2_ref[...], t1,
                                     (((0,), (1,)), ((), ())),
                                     preferred_element_type=jnp.float32)
                     + bc2_ref[...])
        msg_ref[...] = msg
        w_ref[...] = w.reshape(eb)

    return pl.pallas_call(
        body,
        grid=(grid,),
        in_specs=[
            pl.BlockSpec((eb, 2 * C), lambda i: (i, 0)),
            pl.BlockSpec((eb, 2 * C), lambda i: (i, 0)),
            pl.BlockSpec((eb,), lambda i: (i,)),
            pl.BlockSpec((1, 2 * C), lambda i: (0, 0)),
            pl.BlockSpec((2 * C, C), lambda i: (0, 0)),
            pl.BlockSpec((1, C), lambda i: (0, 0)),
            pl.BlockSpec((C, C // 2), lambda i: (0, 0)),
            pl.BlockSpec((1, C // 2), lambda i: (0, 0)),
            pl.BlockSpec((C // 2, 1), lambda i: (0, 0)),
            pl.BlockSpec((1, 1), lambda i: (0, 0)),
        ],
        out_specs=[
            pl.BlockSpec((eb, C), lambda i: (i, 0)),
            pl.BlockSpec((eb,), lambda i: (i,)),
        ],
        out_shape=[
            jax.ShapeDtypeStruct((e_tot, C), jnp.float32),
            jax.ShapeDtypeStruct((e_tot,), jnp.float32),
        ],
    )


def _upd_body(h_ref, agg0_ref, agg1_ref, vx0_ref, vx1_ref, vy0_ref, vy1_ref,
              vz0_ref, vz1_ref, wu1a_ref, wu1b_ref, bu1_ref, wu2_ref,
              bu2_ref, g_ref, bt_ref,
              h_out_ref, vx_ref, vy_ref, vz_ref):
    h = h_ref[...]
    agg = agg0_ref[...] + agg1_ref[...]
    u1 = _silu(jnp.dot(h, wu1a_ref[...], preferred_element_type=jnp.float32)
               + jnp.dot(agg, wu1b_ref[...], preferred_element_type=jnp.float32)
               + bu1_ref[...])
    upd = jnp.dot(u1, wu2_ref[...], preferred_element_type=jnp.float32) + bu2_ref[...]
    x = h + upd
    mu = jnp.mean(x, axis=1, keepdims=True)
    xc = x - mu
    var = jnp.mean(xc * xc, axis=1, keepdims=True)
    h_out_ref[...] = xc * lax.rsqrt(var + 1e-5) * g_ref[...] + bt_ref[...]
    vx_ref[...] = vx0_ref[...] + vx1_ref[...]
    vy_ref[...] = vy0_ref[...] + vy1_ref[...]
    vz_ref[...] = vz0_ref[...] + vz1_ref[...]


def _node_update(h_tgt, agg0, agg1, vx0, vx1, vy0, vy1, vz0, vz1,
                 Wu1a, Wu1b, bu1, Wu2, bu2, gamma, beta):
    grid = N_NODE // _NB
    vec = lambda: pl.BlockSpec((N_NODE,), lambda i: (0,))
    return pl.pallas_call(
        _upd_body,
        grid=(grid,),
        in_specs=[
            pl.BlockSpec((_NB, C), lambda i: (i, 0)),
            pl.BlockSpec((_NB, C), lambda i: (i, 0)),
            pl.BlockSpec((_NB, C), lambda i: (i, 0)),
            vec(), vec(), vec(), vec(), vec(), vec(),
            pl.BlockSpec((C, C), lambda i: (0, 0)),
            pl.BlockSpec((C, C), lambda i: (0, 0)),
            pl.BlockSpec((1, C), lambda i: (0, 0)),
            pl.BlockSpec((C, C), lambda i: (0, 0)),
            pl.BlockSpec((1, C), lambda i: (0, 0)),
            pl.BlockSpec((1, C), lambda i: (0, 0)),
            pl.BlockSpec((1, C), lambda i: (0, 0)),
        ],
        out_specs=[
            pl.BlockSpec((_NB, C), lambda i: (i, 0)),
            vec(), vec(), vec(),
        ],
        out_shape=[
            jax.ShapeDtypeStruct((N_NODE, C), jnp.float32),
            jax.ShapeDtypeStruct((N_NODE,), jnp.float32),
            jax.ShapeDtypeStruct((N_NODE,), jnp.float32),
            jax.ShapeDtypeStruct((N_NODE,), jnp.float32),
        ],
    )(h_tgt, agg0, agg1, vx0, vx1, vy0, vy1, vz0, vz1,
      Wu1a, Wu1b, bu1, Wu2, bu2, gamma, beta)


def kernel(h_src, h_tgt, pos_src, pos_tgt, t_emb_tgt, edge_src, edge_tgt,
           W1, b1, W2, b2, Wc1, bc1, Wc2, bc2, Wu1, bu1, Wu2, bu2,
           gamma, beta):
    w1s = W1[0:C]
    w1t = W1[C:2 * C]
    w1d = W1[2 * C:2 * C + 1]
    w1e = W1[2 * C + 1:]
    A, B = _node_pre(h_src, h_tgt, t_emb_tgt, w1s, w1t, w1e,
                     b1.reshape(1, -1))

    psx, psy, psz = pos_src[:, 0], pos_src[:, 1], pos_src[:, 2]
    ptx, pty, ptz = pos_tgt[:, 0], pos_tgt[:, 1], pos_tgt[:, 2]
    pos = (psx, psy, psz, ptx, pty, ptz)

    gather = _make_gather(_EH, 40)
    geo = _make_geo(_EH)
    mlp = _make_edge_mlp(_EH, 256)
    scat = _make_scatter(_EH, 40)
    vel = _make_vel(_EH, 40)

    agg_p = jnp.zeros((2, N_NODE, C), jnp.float32)
    vx_p = jnp.zeros((2, N_NODE), jnp.float32)
    vy_p = jnp.zeros((2, N_NODE), jnp.float32)
    vz_p = jnp.zeros((2, N_NODE), jnp.float32)

    for p in range(_PHASES):
        es = lax.dynamic_slice_in_dim(edge_src, p * _EH, _EH)
        et = lax.dynamic_slice_in_dim(edge_tgt, p * _EH, _EH)
        a_rows, b_rows = gather(A, B, es, et)
        sq = geo(*pos, es, et)[0]
        msg, w = mlp(a_rows, b_rows, sq, w1d, W2, b2.reshape(1, -1), Wc1,
                     bc1.reshape(1, -1), Wc2, bc2.reshape(1, 1))
        agg_p = scat(msg, et, agg_p)[0]
        vx_p, vy_p, vz_p = vel(w, *pos, es, et, vx_p, vy_p, vz_p)

    h_new, vx, vy, vz = _node_update(
        h_tgt, agg_p[0], agg_p[1], vx_p[0], vx_p[1], vy_p[0], vy_p[1],
        vz_p[0], vz_p[1], Wu1[:C], Wu1[C:], bu1.reshape(1, -1), Wu2,
        bu2.reshape(1, -1), gamma.reshape(1, -1), beta.reshape(1, -1))
    return h_new, jnp.stack([vx, vy, vz], axis=1)


# speedup vs baseline: 1.4119x; 1.4119x over previous
import functools

import jax
import jax.numpy as jnp
from jax import lax
from jax.experimental import pallas as pl
from jax.experimental.pallas import tpu as pltpu
from jax.experimental.pallas import tpu_sc as plsc

N_NODE = 10000
E_TOT = 320000
C = 128

_NB = 1000
_NW = 32

_E1 = 163840
_E2 = E_TOT - _E1

_sc_mesh = functools.partial(
    plsc.VectorSubcoreMesh, core_axis_name="c", subcore_axis_name="s")


def _silu(x):
    return x * jax.nn.sigmoid(x)


def _wid():
    return lax.axis_index("s") * 2 + lax.axis_index("c")


def _make_gather(e_tot, ch):
    epw = e_tot // _NW
    nch = epw // ch
    assert epw * _NW == e_tot and nch * ch == epw and ch % 8 == 0

    def body(a_hbm, b_hbm, es_hbm, et_hbm, a_out, b_out,
             bufs0, bufs1, gsem0, gsem1, osem0, osem1):
        tbase = _wid() * epw

        def fire_g(base, bufs, gsem):
            idx_s, idx_t, ab, bb = bufs
            pltpu.sync_copy(es_hbm.at[pl.ds(base, ch)], idx_s)
            pltpu.sync_copy(et_hbm.at[pl.ds(base, ch)], idx_t)
            pltpu.async_copy(a_hbm.at[idx_s], ab, gsem)
            pltpu.async_copy(b_hbm.at[idx_t], bb, gsem)

        def wait_g(bufs, sem):
            _, _, ab, bb = bufs
            pltpu.make_async_copy(a_hbm.at[pl.ds(0, ch)], ab, sem).wait()
            pltpu.make_async_copy(b_hbm.at[pl.ds(0, ch)], bb, sem).wait()

        def fire_o(base, bufs, sem):
            _, _, ab, bb = bufs
            pltpu.async_copy(ab, a_out.at[pl.ds(base, ch)], sem)
            pltpu.async_copy(bb, b_out.at[pl.ds(base, ch)], sem)

        def wait_o(bufs, sem):
            _, _, ab, bb = bufs
            pltpu.make_async_copy(ab, a_out.at[pl.ds(0, ch)], sem).wait()
            pltpu.make_async_copy(bb, b_out.at[pl.ds(0, ch)], sem).wait()

        fire_g(tbase, bufs0, gsem0)

        def loop(j, carry):
            base0 = tbase + (2 * j) * ch
            base1 = base0 + ch
            base2 = base0 + 2 * ch

            @pl.when(j > 0)
            def _():
                wait_o(bufs1, osem1)

            @pl.when(2 * j + 1 < nch)
            def _():
                fire_g(base1, bufs1, gsem1)

            wait_g(bufs0, gsem0)
            fire_o(base0, bufs0, osem0)

            @pl.when(2 * j + 2 < nch)
            def _():
                wait_o(bufs0, osem0)
                fire_g(base2, bufs0, gsem0)

            @pl.when(2 * j + 1 < nch)
            def _():
                wait_g(bufs1, gsem1)
                fire_o(base1, bufs1, osem1)

            return carry

        lax.fori_loop(0, (nch + 1) // 2, loop, 0)
        wait_o(bufs0, osem0)
        if nch % 2 == 0:
            wait_o(bufs1, osem1)

    bufset = lambda: [
        pltpu.VMEM((ch,), jnp.int32),
        pltpu.VMEM((ch,), jnp.int32),
        pltpu.VMEM((ch, 2 * C), jnp.float32),
        pltpu.VMEM((ch, 2 * C), jnp.float32),
    ]
    return pl.kernel(
        body, mesh=_sc_mesh(),
        out_type=[
            jax.ShapeDtypeStruct((e_tot, 2 * C), jnp.float32),
            jax.ShapeDtypeStruct((e_tot, 2 * C), jnp.float32),
        ],
        scratch_types=[
            bufset(), bufset(),
            pltpu.SemaphoreType.DMA, pltpu.SemaphoreType.DMA,
            pltpu.SemaphoreType.DMA, pltpu.SemaphoreType.DMA,
        ],
    )


def _make_geo(e_tot):
    epw = e_tot // _NW
    assert epw * _NW == e_tot and epw % 8 == 0
    ngrp = (epw + 15) // 16

    def body(psx_h, psy_h, psz_h, ptx_h, pty_h, ptz_h, es_hbm, et_hbm,
             sq_out, ptabs, idx_s, idx_t, sqb):
        tbase = _wid() * epw

        psx_t, psy_t, psz_t, ptx_t, pty_t, ptz_t = ptabs
        pltpu.sync_copy(psx_h, psx_t)
        pltpu.sync_copy(psy_h, psy_t)
        pltpu.sync_copy(psz_h, psz_t)
        pltpu.sync_copy(ptx_h, ptx_t)
        pltpu.sync_copy(pty_h, pty_t)
        pltpu.sync_copy(ptz_h, ptz_t)

        pltpu.sync_copy(es_hbm.at[pl.ds(tbase, epw)], idx_s)
        pltpu.sync_copy(et_hbm.at[pl.ds(tbase, epw)], idx_t)

        def grp(g, carry):
            sl = pl.ds(jnp.minimum(g * 16, epw - 16), 16)
            i_s = idx_s[sl]
            i_t = idx_t[sl]
            dx = (plsc.load_gather(ptx_t, [i_t])
                  - plsc.load_gather(psx_t, [i_s]))
            dy = (plsc.load_gather(pty_t, [i_t])
                  - plsc.load_gather(psy_t, [i_s]))
            dz = (plsc.load_gather(ptz_t, [i_t])
                  - plsc.load_gather(psz_t, [i_s]))
            sqb[sl] = dx * dx + dy * dy + dz * dz
            return carry

        lax.fori_loop(0, ngrp, grp, 0)
        pltpu.sync_copy(sqb, sq_out.at[pl.ds(tbase, epw)])

    return pl.kernel(
        body, mesh=_sc_mesh(),
        compiler_params=pltpu.CompilerParams(needs_layout_passes=False),
        out_type=[jax.ShapeDtypeStruct((e_tot,), jnp.float32)],
        scratch_types=[
            [pltpu.VMEM((N_NODE,), jnp.float32) for _ in range(6)],
            pltpu.VMEM((epw,), jnp.int32),
            pltpu.VMEM((epw,), jnp.int32),
            pltpu.VMEM((epw,), jnp.float32),
        ],
    )


def _make_scatter(e_tot, ch):
    epw = e_tot // _NW
    nch = epw // ch
    assert nch * ch == epw and ch % 8 == 0

    def body(msg_hbm, et_hbm, init_hbm, agg_out,
             bufs0, bufs1, agg_acc, lsem0, lsem1):
        c = lax.axis_index("c")
        s = lax.axis_index("s")
        tbase = (s * 2 + c) * epw

        @pl.when(s == 0)
        def _init():
            pltpu.sync_copy(init_hbm.at[c], agg_acc)

        def fire_l(base, bufs, sem):
            idx_t, mb = bufs
            pltpu.sync_copy(et_hbm.at[pl.ds(base, ch)], idx_t)
            pltpu.async_copy(msg_hbm.at[pl.ds(base, ch)], mb, sem)

        def wait_l(bufs, sem):
            pltpu.make_async_copy(msg_hbm.at[pl.ds(0, ch)], bufs[1],
                                  sem).wait()

        def scat(bufs):
            idx_t, mb = bufs
            pltpu.sync_copy(mb, agg_acc.at[idx_t], add=True)

        plsc.subcore_barrier()
        fire_l(tbase, bufs0, lsem0)

        def loop(j, carry):
            base1 = tbase + (2 * j + 1) * ch
            base2 = tbase + (2 * j + 2) * ch

            @pl.when(2 * j + 1 < nch)
            def _():
                fire_l(base1, bufs1, lsem1)

            wait_l(bufs0, lsem0)
            scat(bufs0)

            @pl.when(2 * j + 2 < nch)
            def _():
                fire_l(base2, bufs0, lsem0)

            @pl.when(2 * j + 1 < nch)
            def _():
                wait_l(bufs1, lsem1)
                scat(bufs1)

            return carry

        lax.fori_loop(0, (nch + 1) // 2, loop, 0)
        plsc.subcore_barrier()

        rows = N_NODE // 16
        rbase = s * rows
        pltpu.sync_copy(agg_acc.at[pl.ds(rbase, rows)],
                        agg_out.at[c].at[pl.ds(rbase, rows)])

    bufset = lambda: [
        pltpu.VMEM((ch,), jnp.int32),
        pltpu.VMEM((ch, C), jnp.float32),
    ]
    return pl.kernel(
        body, mesh=_sc_mesh(),
        compiler_params=pltpu.CompilerParams(use_tc_tiling_on_sc=False),
        out_type=[jax.ShapeDtypeStruct((2, N_NODE, C), jnp.float32)],
        scratch_types=[
            bufset(), bufset(),
            pltpu.VMEM_SHARED((N_NODE, C), jnp.float32),
            pltpu.SemaphoreType.DMA, pltpu.SemaphoreType.DMA,
        ],
    )


def _make_vel(e_tot, ch):
    epw = e_tot // _NW
    nch = epw // ch
    assert nch * ch == epw and ch % 8 == 0

    def body(w_hbm, psx_h, psy_h, psz_h, ptx_h, pty_h, ptz_h,
             es_hbm, et_hbm, ivx_h, ivy_h, ivz_h,
             vx_out, vy_out, vz_out,
             bufs0, bufs1, ptabs, accs, lsem0, lsem1):
        c = lax.axis_index("c")
        s = lax.axis_index("s")
        tbase = (s * 2 + c) * epw
        vx_acc, vy_acc, vz_acc = accs

        psx_t, psy_t, psz_t, ptx_t, pty_t, ptz_t = ptabs
        pltpu.sync_copy(psx_h, psx_t)
        pltpu.sync_copy(psy_h, psy_t)
        pltpu.sync_copy(psz_h, psz_t)
        pltpu.sync_copy(ptx_h, ptx_t)
        pltpu.sync_copy(pty_h, pty_t)
        pltpu.sync_copy(ptz_h, ptz_t)

        @pl.when(s == 0)
        def _init():
            pltpu.sync_copy(ivx_h.at[c], vx_acc)
            pltpu.sync_copy(ivy_h.at[c], vy_acc)
            pltpu.sync_copy(ivz_h.at[c], vz_acc)

        def fire_l(base, bufs, sem):
            idx_s, idx_t, wb, xb, yb, zb = bufs
            pltpu.sync_copy(es_hbm.at[pl.ds(base, ch)], idx_s)
            pltpu.sync_copy(et_hbm.at[pl.ds(base, ch)], idx_t)
            pltpu.async_copy(w_hbm.at[pl.ds(base, ch)], wb, sem)

        def wait_l(bufs, sem):
            pltpu.make_async_copy(w_hbm.at[pl.ds(0, ch)], bufs[2],
                                  sem).wait()

        def scat(bufs):
            idx_s, idx_t, wb, xb, yb, zb = bufs
            for g in range((ch + 15) // 16):
                sl = pl.ds(min(g * 16, ch - 16), 16)
                i_s = idx_s[sl]
                i_t = idx_t[sl]
                wv = wb[sl]
                xb[sl] = wv * (plsc.load_gather(ptx_t, [i_t])
                               - plsc.load_gather(psx_t, [i_s]))
                yb[sl] = wv * (plsc.load_gather(pty_t, [i_t])
                               - plsc.load_gather(psy_t, [i_s]))
                zb[sl] = wv * (plsc.load_gather(ptz_t, [i_t])
                               - plsc.load_gather(psz_t, [i_s]))
            pltpu.sync_copy(xb, vx_acc.at[idx_t], add=True)
            pltpu.sync_copy(yb, vy_acc.at[idx_t], add=True)
            pltpu.sync_copy(zb, vz_acc.at[idx_t], add=True)

        plsc.subcore_barrier()
        fire_l(tbase, bufs0, lsem0)

        def loop(j, carry):
            base1 = tbase + (2 * j + 1) * ch
            base2 = tbase + (2 * j + 2) * ch

            @pl.when(2 * j + 1 < nch)
            def _():
                fire_l(base1, bufs1, lsem1)

            wait_l(bufs0, lsem0)
            scat(bufs0)

            @pl.when(2 * j + 2 < nch)
            def _():
                fire_l(base2, bufs0, lsem0)

            @pl.when(2 * j + 1 < nch)
            def _():
                wait_l(bufs1, lsem1)
                scat(bufs1)

            return carry

        lax.fori_loop(0, (nch + 1) // 2, loop, 0)
        plsc.subcore_barrier()

        rows = N_NODE // 10
        rbase = s * rows

        @pl.when(s < 10)
        def _dump():
            pltpu.sync_copy(vx_acc.at[pl.ds(rbase, rows)],
                            vx_out.at[c].at[pl.ds(rbase, rows)])
            pltpu.sync_copy(vy_acc.at[pl.ds(rbase, rows)],
                            vy_out.at[c].at[pl.ds(rbase, rows)])
            pltpu.sync_copy(vz_acc.at[pl.ds(rbase, rows)],
                            vz_out.at[c].at[pl.ds(rbase, rows)])

    bufset = lambda: [
        pltpu.VMEM((ch,), jnp.int32),
        pltpu.VMEM((ch,), jnp.int32),
        pltpu.VMEM((ch,), jnp.float32),
        pltpu.VMEM((ch,), jnp.float32),
        pltpu.VMEM((ch,), jnp.float32),
        pltpu.VMEM((ch,), jnp.float32),
    ]
    return pl.kernel(
        body, mesh=_sc_mesh(),
        compiler_params=pltpu.CompilerParams(
            use_tc_tiling_on_sc=False, needs_layout_passes=False),
        out_type=[
            jax.ShapeDtypeStruct((2, N_NODE), jnp.float32),
            jax.ShapeDtypeStruct((2, N_NODE), jnp.float32),
            jax.ShapeDtypeStruct((2, N_NODE), jnp.float32),
        ],
        scratch_types=[
            bufset(), bufset(),
            [pltpu.VMEM((N_NODE,), jnp.float32) for _ in range(6)],
            [pltpu.VMEM_SHARED((N_NODE,), jnp.float32) for _ in range(3)],
            pltpu.SemaphoreType.DMA, pltpu.SemaphoreType.DMA,
        ],
    )


def _pre_body(h_src_ref, h_tgt_ref, t_emb_ref, w1s_ref, w1t_ref, w1e_ref,
              b1_ref, a_ref, b_ref):
    a_ref[...] = jnp.dot(h_src_ref[...], w1s_ref[...],
                         preferred_element_type=jnp.float32)
    b_ref[...] = (jnp.dot(h_tgt_ref[...], w1t_ref[...],
                          preferred_element_type=jnp.float32)
                  + jnp.dot(t_emb_ref[...], w1e_ref[...],
                            preferred_element_type=jnp.float32)
                  + b1_ref[...])


def _node_pre(h_src, h_tgt, t_emb, w1s, w1t, w1e, b1):
    grid = N_NODE // _NB
    return pl.pallas_call(
        _pre_body,
        grid=(grid,),
        in_specs=[
            pl.BlockSpec((_NB, C), lambda i: (i, 0)),
            pl.BlockSpec((_NB, C), lambda i: (i, 0)),
            pl.BlockSpec((_NB, 32), lambda i: (i, 0)),
            pl.BlockSpec((C, 2 * C), lambda i: (0, 0)),
            pl.BlockSpec((C, 2 * C), lambda i: (0, 0)),
            pl.BlockSpec((32, 2 * C), lambda i: (0, 0)),
            pl.BlockSpec((1, 2 * C), lambda i: (0, 0)),
        ],
        out_specs=[
            pl.BlockSpec((_NB, 2 * C), lambda i: (i, 0)),
            pl.BlockSpec((_NB, 2 * C), lambda i: (i, 0)),
        ],
        out_shape=[
            jax.ShapeDtypeStruct((N_NODE, 2 * C), jnp.float32),
            jax.ShapeDtypeStruct((N_NODE, 2 * C), jnp.float32),
        ],
    )(h_src, h_tgt, t_emb, w1s, w1t, w1e, b1)


def _make_edge_mlp(e_tot, eb):
    grid = e_tot // eb
    assert grid * eb == e_tot

    def body(a_ref, b_ref, sq_ref, w1d_ref, w2_ref, b2_ref, wc1_ref,
             bc1_ref, wc2_ref, bc2_ref, msg_ref, w_ref):
        sq2 = sq_ref[...].reshape(1, eb)
        sq_term = lax.dot_general(sq2, w1d_ref[...], (((0,), (0,)), ((), ())),
                                  preferred_element_type=jnp.float32)
        pre = a_ref[...] + b_ref[...] + sq_term
        h1 = _silu(pre)
        msg = _silu(jnp.dot(h1, w2_ref[...],
                            preferred_element_type=jnp.float32) + b2_ref[...])
        t1 = _silu(jnp.dot(msg, wc1_ref[...],
                           preferred_element_type=jnp.float32) + bc1_ref[...])
        w = jnp.tanh(lax.dot_general(wc2_ref[...], t1,
                                     (((0,), (1,)), ((), ())),
                                     preferred_element_type=jnp.float32)
                     + bc2_ref[...])
        msg_ref[...] = msg
        w_ref[...] = w.reshape(eb)

    return pl.pallas_call(
        body,
        grid=(grid,),
        in_specs=[
            pl.BlockSpec((eb, 2 * C), lambda i: (i, 0)),
            pl.BlockSpec((eb, 2 * C), lambda i: (i, 0)),
            pl.BlockSpec((eb,), lambda i: (i,)),
            pl.BlockSpec((1, 2 * C), lambda i: (0, 0)),
            pl.BlockSpec((2 * C, C), lambda i: (0, 0)),
            pl.BlockSpec((1, C), lambda i: (0, 0)),
            pl.BlockSpec((C, C // 2), lambda i: (0, 0)),
            pl.BlockSpec((1, C // 2), lambda i: (0, 0)),
            pl.BlockSpec((C // 2, 1), lambda i: (0, 0)),
            pl.BlockSpec((1, 1), lambda i: (0, 0)),
        ],
        out_specs=[
            pl.BlockSpec((eb, C), lambda i: (i, 0)),
            pl.BlockSpec((eb,), lambda i: (i,)),
        ],
        out_shape=[
            jax.ShapeDtypeStruct((e_tot, C), jnp.float32),
            jax.ShapeDtypeStruct((e_tot,), jnp.float32),
        ],
    )


def _upd_body(h_ref, agg0_ref, agg1_ref, vx0_ref, vx1_ref, vy0_ref, vy1_ref,
              vz0_ref, vz1_ref, wu1a_ref, wu1b_ref, bu1_ref, wu2_ref,
              bu2_ref, g_ref, bt_ref,
              h_out_ref, vx_ref, vy_ref, vz_ref):
    h = h_ref[...]
    agg = agg0_ref[...] + agg1_ref[...]
    u1 = _silu(jnp.dot(h, wu1a_ref[...], preferred_element_type=jnp.float32)
               + jnp.dot(agg, wu1b_ref[...], preferred_element_type=jnp.float32)
               + bu1_ref[...])
    upd = jnp.dot(u1, wu2_ref[...], preferred_element_type=jnp.float32) + bu2_ref[...]
    x = h + upd
    mu = jnp.mean(x, axis=1, keepdims=True)
    xc = x - mu
    var = jnp.mean(xc * xc, axis=1, keepdims=True)
    h_out_ref[...] = xc * lax.rsqrt(var + 1e-5) * g_ref[...] + bt_ref[...]
    vx_ref[...] = vx0_ref[...] + vx1_ref[...]
    vy_ref[...] = vy0_ref[...] + vy1_ref[...]
    vz_ref[...] = vz0_ref[...] + vz1_ref[...]


def _node_update(h_tgt, agg0, agg1, vx0, vx1, vy0, vy1, vz0, vz1,
                 Wu1a, Wu1b, bu1, Wu2, bu2, gamma, beta):
    grid = N_NODE // _NB
    vec = lambda: pl.BlockSpec((N_NODE,), lambda i: (0,))
    return pl.pallas_call(
        _upd_body,
        grid=(grid,),
        in_specs=[
            pl.BlockSpec((_NB, C), lambda i: (i, 0)),
            pl.BlockSpec((_NB, C), lambda i: (i, 0)),
            pl.BlockSpec((_NB, C), lambda i: (i, 0)),
            vec(), vec(), vec(), vec(), vec(), vec(),
            pl.BlockSpec((C, C), lambda i: (0, 0)),
            pl.BlockSpec((C, C), lambda i: (0, 0)),
            pl.BlockSpec((1, C), lambda i: (0, 0)),
            pl.BlockSpec((C, C), lambda i: (0, 0)),
            pl.BlockSpec((1, C), lambda i: (0, 0)),
            pl.BlockSpec((1, C), lambda i: (0, 0)),
            pl.BlockSpec((1, C), lambda i: (0, 0)),
        ],
        out_specs=[
            pl.BlockSpec((_NB, C), lambda i: (i, 0)),
            vec(), vec(), vec(),
        ],
        out_shape=[
            jax.ShapeDtypeStruct((N_NODE, C), jnp.float32),
            jax.ShapeDtypeStruct((N_NODE,), jnp.float32),
            jax.ShapeDtypeStruct((N_NODE,), jnp.float32),
            jax.ShapeDtypeStruct((N_NODE,), jnp.float32),
        ],
    )(h_tgt, agg0, agg1, vx0, vx1, vy0, vy1, vz0, vz1,
      Wu1a, Wu1b, bu1, Wu2, bu2, gamma, beta)


def kernel(h_src, h_tgt, pos_src, pos_tgt, t_emb_tgt, edge_src, edge_tgt,
           W1, b1, W2, b2, Wc1, bc1, Wc2, bc2, Wu1, bu1, Wu2, bu2,
           gamma, beta):
    w1s = W1[0:C]
    w1t = W1[C:2 * C]
    w1d = W1[2 * C:2 * C + 1]
    w1e = W1[2 * C + 1:]
    A, B = _node_pre(h_src, h_tgt, t_emb_tgt, w1s, w1t, w1e,
                     b1.reshape(1, -1))

    psx, psy, psz = pos_src[:, 0], pos_src[:, 1], pos_src[:, 2]
    ptx, pty, ptz = pos_tgt[:, 0], pos_tgt[:, 1], pos_tgt[:, 2]
    pos = (psx, psy, psz, ptx, pty, ptz)

    agg_p = jnp.zeros((2, N_NODE, C), jnp.float32)
    vx_p = jnp.zeros((2, N_NODE), jnp.float32)
    vy_p = jnp.zeros((2, N_NODE), jnp.float32)
    vz_p = jnp.zeros((2, N_NODE), jnp.float32)

    for e_lo, e_n, ch in ((0, _E1, 128), (_E1, _E2, 80)):
        es = edge_src[e_lo:e_lo + e_n]
        et = edge_tgt[e_lo:e_lo + e_n]
        a_rows, b_rows = _make_gather(e_n, 80)(A, B, es, et)
        sq = _make_geo(e_n)(*pos, es, et)[0]
        msg, w = _make_edge_mlp(e_n, 512)(
            a_rows, b_rows, sq, w1d, W2, b2.reshape(1, -1), Wc1,
            bc1.reshape(1, -1), Wc2, bc2.reshape(1, 1))
        agg_p = _make_scatter(e_n, ch)(msg, et, agg_p)[0]
        vx_p, vy_p, vz_p = _make_vel(e_n, ch)(w, *pos, es, et,
                                              vx_p, vy_p, vz_p)

    h_new, vx, vy, vz = _node_update(
        h_tgt, agg_p[0], agg_p[1], vx_p[0], vx_p[1], vy_p[0], vy_p[1],
        vz_p[0], vz_p[1], Wu1[:C], Wu1[C:], bu1.reshape(1, -1), Wu2,
        bu2.reshape(1, -1), gamma.reshape(1, -1), beta.reshape(1, -1))
    return h_new, jnp.stack([vx, vy, vz], axis=1)


# speedup vs baseline: 1.7345x; 1.2285x over previous
import functools

import jax
import jax.numpy as jnp
from jax import lax
from jax.experimental import pallas as pl
from jax.experimental.pallas import tpu as pltpu
from jax.experimental.pallas import tpu_sc as plsc

N_NODE = 10000
E_TOT = 320000
C = 128

_NB = 1000
_NW = 32

_E1 = 163840
_E2 = E_TOT - _E1

_sc_mesh = functools.partial(
    plsc.VectorSubcoreMesh, core_axis_name="c", subcore_axis_name="s")


def _silu(x):
    return x * jax.nn.sigmoid(x)


def _wid():
    return lax.axis_index("s") * 2 + lax.axis_index("c")


def _make_gather(e_tot, ch):
    epw = e_tot // _NW
    nch = epw // ch
    assert epw * _NW == e_tot and nch * ch == epw and ch % 8 == 0

    def body(a_hbm, b_hbm, es_hbm, et_hbm, a_out, b_out,
             bufs0, bufs1, gsem0, gsem1, osem0, osem1):
        tbase = _wid() * epw

        def fire_g(base, bufs, gsem):
            idx_s, idx_t, ab, bb = bufs
            pltpu.sync_copy(es_hbm.at[pl.ds(base, ch)], idx_s)
            pltpu.sync_copy(et_hbm.at[pl.ds(base, ch)], idx_t)
            pltpu.async_copy(a_hbm.at[idx_s], ab, gsem)
            pltpu.async_copy(b_hbm.at[idx_t], bb, gsem)

        def wait_g(bufs, sem):
            _, _, ab, bb = bufs
            pltpu.make_async_copy(a_hbm.at[pl.ds(0, ch)], ab, sem).wait()
            pltpu.make_async_copy(b_hbm.at[pl.ds(0, ch)], bb, sem).wait()

        def fire_o(base, bufs, sem):
            _, _, ab, bb = bufs
            pltpu.async_copy(ab, a_out.at[pl.ds(base, ch)], sem)
            pltpu.async_copy(bb, b_out.at[pl.ds(base, ch)], sem)

        def wait_o(bufs, sem):
            _, _, ab, bb = bufs
            pltpu.make_async_copy(ab, a_out.at[pl.ds(0, ch)], sem).wait()
            pltpu.make_async_copy(bb, b_out.at[pl.ds(0, ch)], sem).wait()

        fire_g(tbase, bufs0, gsem0)

        def loop(j, carry):
            base0 = tbase + (2 * j) * ch
            base1 = base0 + ch
            base2 = base0 + 2 * ch

            @pl.when(j > 0)
            def _():
                wait_o(bufs1, osem1)

            @pl.when(2 * j + 1 < nch)
            def _():
                fire_g(base1, bufs1, gsem1)

            wait_g(bufs0, gsem0)
            fire_o(base0, bufs0, osem0)

            @pl.when(2 * j + 2 < nch)
            def _():
                wait_o(bufs0, osem0)
                fire_g(base2, bufs0, gsem0)

            @pl.when(2 * j + 1 < nch)
            def _():
                wait_g(bufs1, gsem1)
                fire_o(base1, bufs1, osem1)

            return carry

        lax.fori_loop(0, (nch + 1) // 2, loop, 0)
        wait_o(bufs0, osem0)
        if nch % 2 == 0:
            wait_o(bufs1, osem1)

    bufset = lambda: [
        pltpu.VMEM((ch,), jnp.int32),
        pltpu.VMEM((ch,), jnp.int32),
        pltpu.VMEM((ch, C), jnp.int32),
        pltpu.VMEM((ch, C), jnp.int32),
    ]
    return pl.kernel(
        body, mesh=_sc_mesh(),
        out_type=[
            jax.ShapeDtypeStruct((e_tot, C), jnp.int32),
            jax.ShapeDtypeStruct((e_tot, C), jnp.int32),
        ],
        scratch_types=[
            bufset(), bufset(),
            pltpu.SemaphoreType.DMA, pltpu.SemaphoreType.DMA,
            pltpu.SemaphoreType.DMA, pltpu.SemaphoreType.DMA,
        ],
    )


def _make_geo(e_tot):
    epw = e_tot // _NW
    assert epw * _NW == e_tot and epw % 8 == 0
    ngrp = (epw + 15) // 16

    def body(psx_h, psy_h, psz_h, ptx_h, pty_h, ptz_h, es_hbm, et_hbm,
             sq_out, ptabs, idx_s, idx_t, sqb):
        tbase = _wid() * epw

        psx_t, psy_t, psz_t, ptx_t, pty_t, ptz_t = ptabs
        pltpu.sync_copy(psx_h, psx_t)
        pltpu.sync_copy(psy_h, psy_t)
        pltpu.sync_copy(psz_h, psz_t)
        pltpu.sync_copy(ptx_h, ptx_t)
        pltpu.sync_copy(pty_h, pty_t)
        pltpu.sync_copy(ptz_h, ptz_t)

        pltpu.sync_copy(es_hbm.at[pl.ds(tbase, epw)], idx_s)
        pltpu.sync_copy(et_hbm.at[pl.ds(tbase, epw)], idx_t)

        def grp(g, carry):
            sl = pl.ds(jnp.minimum(g * 16, epw - 16), 16)
            i_s = idx_s[sl]
            i_t = idx_t[sl]
            dx = (plsc.load_gather(ptx_t, [i_t])
                  - plsc.load_gather(psx_t, [i_s]))
            dy = (plsc.load_gather(pty_t, [i_t])
                  - plsc.load_gather(psy_t, [i_s]))
            dz = (plsc.load_gather(ptz_t, [i_t])
                  - plsc.load_gather(psz_t, [i_s]))
            sqb[sl] = dx * dx + dy * dy + dz * dz
            return carry

        lax.fori_loop(0, ngrp, grp, 0)
        pltpu.sync_copy(sqb, sq_out.at[pl.ds(tbase, epw)])

    return pl.kernel(
        body, mesh=_sc_mesh(),
        compiler_params=pltpu.CompilerParams(needs_layout_passes=False),
        out_type=[jax.ShapeDtypeStruct((e_tot,), jnp.float32)],
        scratch_types=[
            [pltpu.VMEM((N_NODE,), jnp.float32) for _ in range(6)],
            pltpu.VMEM((epw,), jnp.int32),
            pltpu.VMEM((epw,), jnp.int32),
            pltpu.VMEM((epw,), jnp.float32),
        ],
    )


def _make_scatter(e_tot, ch):
    epw = e_tot // _NW
    nch = epw // ch
    assert nch * ch == epw and ch % 8 == 0

    def body(msg_hbm, et_hbm, init_hbm, agg_out,
             bufs0, bufs1, agg_acc, lsem0, lsem1):
        c = lax.axis_index("c")
        s = lax.axis_index("s")
        tbase = (s * 2 + c) * epw

        @pl.when(s == 0)
        def _init():
            pltpu.sync_copy(init_hbm.at[c], agg_acc)

        def fire_l(base, bufs, sem):
            idx_t, mb = bufs
            pltpu.sync_copy(et_hbm.at[pl.ds(base, ch)], idx_t)
            pltpu.async_copy(msg_hbm.at[pl.ds(base, ch)], mb, sem)

        def wait_l(bufs, sem):
            pltpu.make_async_copy(msg_hbm.at[pl.ds(0, ch)], bufs[1],
                                  sem).wait()

        def scat(bufs):
            idx_t, mb = bufs
            pltpu.sync_copy(mb, agg_acc.at[idx_t], add=True)

        plsc.subcore_barrier()
        fire_l(tbase, bufs0, lsem0)

        def loop(j, carry):
            base1 = tbase + (2 * j + 1) * ch
            base2 = tbase + (2 * j + 2) * ch

            @pl.when(2 * j + 1 < nch)
            def _():
                fire_l(base1, bufs1, lsem1)

            wait_l(bufs0, lsem0)
            scat(bufs0)

            @pl.when(2 * j + 2 < nch)
            def _():
                fire_l(base2, bufs0, lsem0)

            @pl.when(2 * j + 1 < nch)
            def _():
                wait_l(bufs1, lsem1)
                scat(bufs1)

            return carry

        lax.fori_loop(0, (nch + 1) // 2, loop, 0)
        plsc.subcore_barrier()

        rows = N_NODE // 16
        rbase = s * rows
        pltpu.sync_copy(agg_acc.at[pl.ds(rbase, rows)],
                        agg_out.at[c].at[pl.ds(rbase, rows)])

    bufset = lambda: [
        pltpu.VMEM((ch,), jnp.int32),
        pltpu.VMEM((ch, C), jnp.float32),
    ]
    return pl.kernel(
        body, mesh=_sc_mesh(),
        compiler_params=pltpu.CompilerParams(use_tc_tiling_on_sc=False),
        out_type=[jax.ShapeDtypeStruct((2, N_NODE, C), jnp.float32)],
        scratch_types=[
            bufset(), bufset(),
            pltpu.VMEM_SHARED((N_NODE, C), jnp.float32),
            pltpu.SemaphoreType.DMA, pltpu.SemaphoreType.DMA,
        ],
    )


def _make_vel(e_tot, ch):
    epw = e_tot // _NW
    nch = epw // ch
    assert nch * ch == epw and ch % 8 == 0

    def body(w_hbm, psx_h, psy_h, psz_h, ptx_h, pty_h, ptz_h,
             es_hbm, et_hbm, ivx_h, ivy_h, ivz_h,
             vx_out, vy_out, vz_out,
             bufs0, bufs1, ptabs, accs, lsem0, lsem1):
        c = lax.axis_index("c")
        s = lax.axis_index("s")
        tbase = (s * 2 + c) * epw
        vx_acc, vy_acc, vz_acc = accs

        psx_t, psy_t, psz_t, ptx_t, pty_t, ptz_t = ptabs
        pltpu.sync_copy(psx_h, psx_t)
        pltpu.sync_copy(psy_h, psy_t)
        pltpu.sync_copy(psz_h, psz_t)
        pltpu.sync_copy(ptx_h, ptx_t)
        pltpu.sync_copy(pty_h, pty_t)
        pltpu.sync_copy(ptz_h, ptz_t)

        @pl.when(s == 0)
        def _init():
            pltpu.sync_copy(ivx_h.at[c], vx_acc)
            pltpu.sync_copy(ivy_h.at[c], vy_acc)
            pltpu.sync_copy(ivz_h.at[c], vz_acc)

        def fire_l(base, bufs, sem):
            idx_s, idx_t, wb, xb, yb, zb = bufs
            pltpu.sync_copy(es_hbm.at[pl.ds(base, ch)], idx_s)
            pltpu.sync_copy(et_hbm.at[pl.ds(base, ch)], idx_t)
            pltpu.async_copy(w_hbm.at[pl.ds(base, ch)], wb, sem)

        def wait_l(bufs, sem):
            pltpu.make_async_copy(w_hbm.at[pl.ds(0, ch)], bufs[2],
                                  sem).wait()

        def scat(bufs):
            idx_s, idx_t, wb, xb, yb, zb = bufs
            for g in range((ch + 15) // 16):
                sl = pl.ds(min(g * 16, ch - 16), 16)
                i_s = idx_s[sl]
                i_t = idx_t[sl]
                wv = wb[sl]
                xb[sl] = wv * (plsc.load_gather(ptx_t, [i_t])
                               - plsc.load_gather(psx_t, [i_s]))
                yb[sl] = wv * (plsc.load_gather(pty_t, [i_t])
                               - plsc.load_gather(psy_t, [i_s]))
                zb[sl] = wv * (plsc.load_gather(ptz_t, [i_t])
                               - plsc.load_gather(psz_t, [i_s]))
            pltpu.sync_copy(xb, vx_acc.at[idx_t], add=True)
            pltpu.sync_copy(yb, vy_acc.at[idx_t], add=True)
            pltpu.sync_copy(zb, vz_acc.at[idx_t], add=True)

        plsc.subcore_barrier()
        fire_l(tbase, bufs0, lsem0)

        def loop(j, carry):
            base1 = tbase + (2 * j + 1) * ch
            base2 = tbase + (2 * j + 2) * ch

            @pl.when(2 * j + 1 < nch)
            def _():
                fire_l(base1, bufs1, lsem1)

            wait_l(bufs0, lsem0)
            scat(bufs0)

            @pl.when(2 * j + 2 < nch)
            def _():
                fire_l(base2, bufs0, lsem0)

            @pl.when(2 * j + 1 < nch)
            def _():
                wait_l(bufs1, lsem1)
                scat(bufs1)

            return carry

        lax.fori_loop(0, (nch + 1) // 2, loop, 0)
        plsc.subcore_barrier()

        rows = N_NODE // 10
        rbase = s * rows

        @pl.when(s < 10)
        def _dump():
            pltpu.sync_copy(vx_acc.at[pl.ds(rbase, rows)],
                            vx_out.at[c].at[pl.ds(rbase, rows)])
            pltpu.sync_copy(vy_acc.at[pl.ds(rbase, rows)],
                            vy_out.at[c].at[pl.ds(rbase, rows)])
            pltpu.sync_copy(vz_acc.at[pl.ds(rbase, rows)],
                            vz_out.at[c].at[pl.ds(rbase, rows)])

    bufset = lambda: [
        pltpu.VMEM((ch,), jnp.int32),
        pltpu.VMEM((ch,), jnp.int32),
        pltpu.VMEM((ch,), jnp.float32),
        pltpu.VMEM((ch,), jnp.float32),
        pltpu.VMEM((ch,), jnp.float32),
        pltpu.VMEM((ch,), jnp.float32),
    ]
    return pl.kernel(
        body, mesh=_sc_mesh(),
        compiler_params=pltpu.CompilerParams(
            use_tc_tiling_on_sc=False, needs_layout_passes=False),
        out_type=[
            jax.ShapeDtypeStruct((2, N_NODE), jnp.float32),
            jax.ShapeDtypeStruct((2, N_NODE), jnp.float32),
            jax.ShapeDtypeStruct((2, N_NODE), jnp.float32),
        ],
        scratch_types=[
            bufset(), bufset(),
            [pltpu.VMEM((N_NODE,), jnp.float32) for _ in range(6)],
            [pltpu.VMEM_SHARED((N_NODE,), jnp.float32) for _ in range(3)],
            pltpu.SemaphoreType.DMA, pltpu.SemaphoreType.DMA,
        ],
    )


def _pre_body(h_src_ref, h_tgt_ref, t_emb_ref, w1s_ref, w1t_ref, w1e_ref,
              b1_ref, a_ref, b_ref):
    def pack(x0, x1):
        b0 = lax.bitcast_convert_type(x0, jnp.int32)
        b1 = lax.bitcast_convert_type(x1, jnp.int32)
        lo = lax.shift_right_logical(b0 + 0x8000, 16)
        hi = (b1 + 0x8000) & jnp.int32(-65536)
        return lo | hi

    halves = []
    for half in range(2):
        cs = pl.ds(half * C, C)
        a = jnp.dot(h_src_ref[...], w1s_ref[:, cs],
                    preferred_element_type=jnp.float32)
        b = (jnp.dot(h_tgt_ref[...], w1t_ref[:, cs],
                     preferred_element_type=jnp.float32)
             + jnp.dot(t_emb_ref[...], w1e_ref[:, cs],
                       preferred_element_type=jnp.float32)
             + b1_ref[:, cs])
        halves.append((a, b))
    a_ref[...] = pack(halves[0][0], halves[1][0])
    b_ref[...] = pack(halves[0][1], halves[1][1])


def _node_pre(h_src, h_tgt, t_emb, w1s, w1t, w1e, b1):
    grid = N_NODE // _NB
    return pl.pallas_call(
        _pre_body,
        grid=(grid,),
        in_specs=[
            pl.BlockSpec((_NB, C), lambda i: (i, 0)),
            pl.BlockSpec((_NB, C), lambda i: (i, 0)),
            pl.BlockSpec((_NB, 32), lambda i: (i, 0)),
            pl.BlockSpec((C, 2 * C), lambda i: (0, 0)),
            pl.BlockSpec((C, 2 * C), lambda i: (0, 0)),
            pl.BlockSpec((32, 2 * C), lambda i: (0, 0)),
            pl.BlockSpec((1, 2 * C), lambda i: (0, 0)),
        ],
        out_specs=[
            pl.BlockSpec((_NB, C), lambda i: (i, 0)),
            pl.BlockSpec((_NB, C), lambda i: (i, 0)),
        ],
        out_shape=[
            jax.ShapeDtypeStruct((N_NODE, C), jnp.int32),
            jax.ShapeDtypeStruct((N_NODE, C), jnp.int32),
        ],
    )(h_src, h_tgt, t_emb, w1s, w1t, w1e, b1)


def _make_edge_mlp(e_tot, eb):
    grid = e_tot // eb
    assert grid * eb == e_tot

    def body(a_ref, b_ref, sq_ref, w1d_ref, w2_ref, b2_ref, wc1_ref,
             bc1_ref, wc2_ref, bc2_ref, msg_ref, w_ref):
        sq2 = sq_ref[...].reshape(1, eb)
        aw = a_ref[...]
        bw = b_ref[...]

        def unpack(wrd, half):
            if half == 0:
                return lax.bitcast_convert_type(
                    lax.shift_left(wrd, 16), jnp.float32)
            return lax.bitcast_convert_type(
                wrd & jnp.int32(-65536), jnp.float32)

        acc = b2_ref[...]
        for half in range(2):
            cs = pl.ds(half * C, C)
            sq_term = lax.dot_general(
                sq2, w1d_ref[:, cs], (((0,), (0,)), ((), ())),
                preferred_element_type=jnp.float32)
            pre = unpack(aw, half) + unpack(bw, half) + sq_term
            acc = acc + jnp.dot(_silu(pre), w2_ref[cs, :],
                                preferred_element_type=jnp.float32)
        msg = _silu(acc)
        t1 = _silu(jnp.dot(msg, wc1_ref[...],
                           preferred_element_type=jnp.float32) + bc1_ref[...])
        w = jnp.tanh(lax.dot_general(wc2_ref[...], t1,
                                     (((0,), (1,)), ((), ())),
                                     preferred_element_type=jnp.float32)
                     + bc2_ref[...])
        msg_ref[...] = msg
        w_ref[...] = w.reshape(eb)

    return pl.pallas_call(
        body,
        grid=(grid,),
        in_specs=[
            pl.BlockSpec((eb, C), lambda i: (i, 0)),
            pl.BlockSpec((eb, C), lambda i: (i, 0)),
            pl.BlockSpec((eb,), lambda i: (i,)),
            pl.BlockSpec((1, 2 * C), lambda i: (0, 0)),
            pl.BlockSpec((2 * C, C), lambda i: (0, 0)),
            pl.BlockSpec((1, C), lambda i: (0, 0)),
            pl.BlockSpec((C, C // 2), lambda i: (0, 0)),
            pl.BlockSpec((1, C // 2), lambda i: (0, 0)),
            pl.BlockSpec((C // 2, 1), lambda i: (0, 0)),
            pl.BlockSpec((1, 1), lambda i: (0, 0)),
        ],
        out_specs=[
            pl.BlockSpec((eb, C), lambda i: (i, 0)),
            pl.BlockSpec((eb,), lambda i: (i,)),
        ],
        out_shape=[
            jax.ShapeDtypeStruct((e_tot, C), jnp.float32),
            jax.ShapeDtypeStruct((e_tot,), jnp.float32),
        ],
    )


def _upd_body(h_ref, agg0_ref, agg1_ref, vx0_ref, vx1_ref, vy0_ref, vy1_ref,
              vz0_ref, vz1_ref, wu1a_ref, wu1b_ref, bu1_ref, wu2_ref,
              bu2_ref, g_ref, bt_ref,
              h_out_ref, vx_ref, vy_ref, vz_ref):
    h = h_ref[...]
    agg = agg0_ref[...] + agg1_ref[...]
    u1 = _silu(jnp.dot(h, wu1a_ref[...], preferred_element_type=jnp.float32)
               + jnp.dot(agg, wu1b_ref[...], preferred_element_type=jnp.float32)
               + bu1_ref[...])
    upd = jnp.dot(u1, wu2_ref[...], preferred_element_type=jnp.float32) + bu2_ref[...]
    x = h + upd
    mu = jnp.mean(x, axis=1, keepdims=True)
    xc = x - mu
    var = jnp.mean(xc * xc, axis=1, keepdims=True)
    h_out_ref[...] = xc * lax.rsqrt(var + 1e-5) * g_ref[...] + bt_ref[...]
    vx_ref[...] = vx0_ref[...] + vx1_ref[...]
    vy_ref[...] = vy0_ref[...] + vy1_ref[...]
    vz_ref[...] = vz0_ref[...] + vz1_ref[...]


def _node_update(h_tgt, agg0, agg1, vx0, vx1, vy0, vy1, vz0, vz1,
                 Wu1a, Wu1b, bu1, Wu2, bu2, gamma, beta):
    grid = N_NODE // _NB
    vec = lambda: pl.BlockSpec((N_NODE,), lambda i: (0,))
    return pl.pallas_call(
        _upd_body,
        grid=(grid,),
        in_specs=[
            pl.BlockSpec((_NB, C), lambda i: (i, 0)),
            pl.BlockSpec((_NB, C), lambda i: (i, 0)),
            pl.BlockSpec((_NB, C), lambda i: (i, 0)),
            vec(), vec(), vec(), vec(), vec(), vec(),
            pl.BlockSpec((C, C), lambda i: (0, 0)),
            pl.BlockSpec((C, C), lambda i: (0, 0)),
            pl.BlockSpec((1, C), lambda i: (0, 0)),
            pl.BlockSpec((C, C), lambda i: (0, 0)),
            pl.BlockSpec((1, C), lambda i: (0, 0)),
            pl.BlockSpec((1, C), lambda i: (0, 0)),
            pl.BlockSpec((1, C), lambda i: (0, 0)),
        ],
        out_specs=[
            pl.BlockSpec((_NB, C), lambda i: (i, 0)),
            vec(), vec(), vec(),
        ],
        out_shape=[
            jax.ShapeDtypeStruct((N_NODE, C), jnp.float32),
            jax.ShapeDtypeStruct((N_NODE,), jnp.float32),
            jax.ShapeDtypeStruct((N_NODE,), jnp.float32),
            jax.ShapeDtypeStruct((N_NODE,), jnp.float32),
        ],
    )(h_tgt, agg0, agg1, vx0, vx1, vy0, vy1, vz0, vz1,
      Wu1a, Wu1b, bu1, Wu2, bu2, gamma, beta)


def kernel(h_src, h_tgt, pos_src, pos_tgt, t_emb_tgt, edge_src, edge_tgt,
           W1, b1, W2, b2, Wc1, bc1, Wc2, bc2, Wu1, bu1, Wu2, bu2,
           gamma, beta):
    w1s = W1[0:C]
    w1t = W1[C:2 * C]
    w1d = W1[2 * C:2 * C + 1]
    w1e = W1[2 * C + 1:]
    A, B = _node_pre(h_src, h_tgt, t_emb_tgt, w1s, w1t, w1e,
                     b1.reshape(1, -1))

    psx, psy, psz = pos_src[:, 0], pos_src[:, 1], pos_src[:, 2]
    ptx, pty, ptz = pos_tgt[:, 0], pos_tgt[:, 1], pos_tgt[:, 2]
    pos = (psx, psy, psz, ptx, pty, ptz)

    agg_p = jnp.zeros((2, N_NODE, C), jnp.float32)
    vx_p = jnp.zeros((2, N_NODE), jnp.float32)
    vy_p = jnp.zeros((2, N_NODE), jnp.float32)
    vz_p = jnp.zeros((2, N_NODE), jnp.float32)

    for e_lo, e_n, ch in ((0, _E1, 128), (_E1, _E2, 80)):
        es = edge_src[e_lo:e_lo + e_n]
        et = edge_tgt[e_lo:e_lo + e_n]
        a_rows, b_rows = _make_gather(e_n, 80)(A, B, es, et)
        sq = _make_geo(e_n)(*pos, es, et)[0]
        msg, w = _make_edge_mlp(e_n, 512)(
            a_rows, b_rows, sq, w1d, W2, b2.reshape(1, -1), Wc1,
            bc1.reshape(1, -1), Wc2, bc2.reshape(1, 1))
        agg_p = _make_scatter(e_n, ch)(msg, et, agg_p)[0]
        vx_p, vy_p, vz_p = _make_vel(e_n, ch)(w, *pos, es, et,
                                              vx_p, vy_p, vz_p)

    h_new, vx, vy, vz = _node_update(
        h_tgt, agg_p[0], agg_p[1], vx_p[0], vx_p[1], vy_p[0], vy_p[1],
        vz_p[0], vz_p[1], Wu1[:C], Wu1[C:], bu1.reshape(1, -1), Wu2,
        bu2.reshape(1, -1), gamma.reshape(1, -1), beta.reshape(1, -1))
    return h_new, jnp.stack([vx, vy, vz], axis=1)
